# baseline (speedup 1.0000x reference)
import functools

import jax
import jax.numpy as jnp
from jax import lax
from jax.experimental import pallas as pl
from jax.experimental.pallas import tpu as pltpu
from jax.experimental.pallas import tpu_sc as plsc

K = 16
RB = 128
IDX_BITS = 11
IDX_MASK = (1 << IDX_BITS) - 1


def _knn_topk_body(x_ref, f_ref, xn_ref, idx_ref, xcrd_ref, xcrdt_ref):
    b = pl.program_id(0)
    r = pl.program_id(1)
    N = x_ref.shape[1]
    D = x_ref.shape[2]

    @pl.when(r == 0)
    def _prep():
        xb = x_ref[0]
        fmask = f_ref[0] > 0.1
        x_crd = jnp.where(fmask, 0.0, xb)
        x_ftr = jnp.where(fmask, xb, 0.0)
        xc = jnp.concatenate([x_crd, x_ftr], axis=1)
        m = jnp.sum(xc, axis=0, keepdims=True) / N
        var = jnp.sum((xc - m) ** 2, axis=0, keepdims=True) / (N - 1.0)
        s = jnp.sqrt(var)
        xn = jnp.clip((xc - m) / (s + 1e-5), -10.0, 10.0)
        xn_ref[0] = xn
        xcrd_ref[...] = x_crd
        xcrdt_ref[...] = x_crd.T

    base = r * RB
    a = xcrd_ref[pl.ds(base, RB), :]
    xt = xcrdt_ref[...]
    sqa = jnp.sum(a * a, axis=1, keepdims=True)
    sqf = jnp.sum(xt * xt, axis=0, keepdims=True)
    d2 = sqa + sqf - 2.0 * jnp.dot(a, xt, preferred_element_type=jnp.float32)
    key = jnp.maximum(d2, 0.0)

    GW = 128
    NG = N // GW
    D1 = 8
    big = jnp.float32(3e38)
    iota_g = lax.broadcasted_iota(jnp.int32, (RB, GW), 1).astype(jnp.float32)
    gw_f = jnp.float32(GW)

    groups = [key[:, g * GW:(g + 1) * GW] for g in range(NG)]

    def extract_rounds(groups, depth):
        pv, pi = [], []
        for _ in range(depth):
            for g in range(NG):
                kg = groups[g]
                mn = jnp.min(kg, axis=1, keepdims=True)
                am = jnp.min(jnp.where(kg == mn, iota_g, gw_f), axis=1,
                             keepdims=True)
                groups[g] = jnp.where(iota_g == am, big, kg)
                pv.append(mn)
                pi.append(am + jnp.float32(g * GW))
        return groups, pv, pi

    def pool_topk(pv_cols, pi_cols):
        pv = jnp.concatenate(pv_cols, axis=1)
        pi = jnp.concatenate(pi_cols, axis=1)
        sel_i, sel_v = [], None
        for _ in range(K):
            mn = jnp.min(pv, axis=1, keepdims=True)
            gi = jnp.min(jnp.where(pv == mn, pi, big), axis=1, keepdims=True)
            pv = jnp.where(pi == gi, big, pv)
            sel_i.append(gi)
            sel_v = mn
        return jnp.concatenate(sel_i, axis=1), sel_v

    groups, pv1, pi1 = extract_rounds(groups, D1)
    idx_f, v16 = pool_topk(pv1, pi1)
    idx_ref[0] = idx_f.astype(jnp.int32) + b * N

    last_round = jnp.concatenate(pv1[-NG:], axis=1)
    violated = jnp.any(last_round <= v16)

    @pl.when(violated)
    def _repair():
        _, pv2, pi2 = extract_rounds(list(groups), K - D1)
        idx2, _ = pool_topk(pv1 + pv2, pi1 + pi2)
        idx_ref[0] = idx2.astype(jnp.int32) + b * N


def _knn_topk(x, features):
    B, N, D = x.shape
    grid = (B, N // RB)
    xn, idxg = pl.pallas_call(
        _knn_topk_body,
        grid=grid,
        in_specs=[
            pl.BlockSpec((1, N, D), lambda b, r: (b, 0, 0)),
            pl.BlockSpec((1, 1, D), lambda b, r: (b, 0, 0)),
        ],
        out_specs=[
            pl.BlockSpec((1, N, 2 * D), lambda b, r: (b, 0, 0)),
            pl.BlockSpec((1, RB, K), lambda b, r: (b, r, 0)),
        ],
        out_shape=[
            jax.ShapeDtypeStruct((B, N, 2 * D), jnp.float32),
            jax.ShapeDtypeStruct((B, N, K), jnp.int32),
        ],
        scratch_shapes=[
            pltpu.VMEM((N, D), jnp.float32),
            pltpu.VMEM((D, N), jnp.float32),
        ],
    )(x, features.reshape(B, 1, D))
    return xn, idxg


def _make_sc_gather(T, D, Btot):
    info = plsc.get_sparse_core_info()
    NW = info.num_cores * info.num_subcores
    CH = 128
    RPB = 8
    bpw = Btot // NW
    nrounds = bpw // (CH * RPB)
    assert bpw % (CH * RPB) == 0 and nrounds >= 2
    mesh = plsc.VectorSubcoreMesh(core_axis_name="c", subcore_axis_name="s")
    rpr = CH * RPB

    @functools.partial(
        pl.kernel,
        mesh=mesh,
        compiler_params=pltpu.CompilerParams(use_tc_tiling_on_sc=False),
        out_type=jax.ShapeDtypeStruct((Btot, D), jnp.float32),
        scratch_types=[
            pltpu.VMEM((bpw // CH, CH), jnp.int32),
            pltpu.VMEM((2, rpr, D), jnp.float32),
            pltpu.SemaphoreType.DMA,
            pltpu.SemaphoreType.DMA,
        ],
    )
    def gk(table_hbm, idx_hbm, out_hbm, idx_v, rows_v, sem0, sem1):
        wid = lax.axis_index("s") * info.num_cores + lax.axis_index("c")
        row0 = wid * (bpw // CH)
        pltpu.sync_copy(idx_hbm.at[pl.ds(row0, bpw // CH)], idx_v)
        base = wid * bpw
        sems = (sem0, sem1)

        def fire(r, buf):
            sem = sems[buf]
            return [
                pltpu.async_copy(
                    table_hbm.at[idx_v.at[r * RPB + j]],
                    rows_v.at[(buf, pl.ds(j * CH, CH))],
                    sem,
                )
                for j in range(RPB)
            ]

        def drain_and_store(r, buf, copies):
            for c in copies:
                c.wait()
            pltpu.sync_copy(rows_v.at[buf],
                            out_hbm.at[pl.ds(base + r * rpr, rpr)])

        prev = fire(0, 0)
        for r in range(1, nrounds):
            cur = fire(r, r % 2)
            drain_and_store(r - 1, (r - 1) % 2, prev)
            prev = cur
        drain_and_store(nrounds - 1, (nrounds - 1) % 2, prev)

    return gk


def _ff_glu_body(flat_ref, xn_ref, w_ref, b_ref, out_ref):
    W1 = w_ref[...]
    IN = W1.shape[0]
    M = W1.shape[1] // 2
    D2 = xn_ref.shape[1]
    half = D2 // 2
    w1c = w_ref[pl.ds(0, half), :]
    for k in range(1, IN // D2):
        w1c = w1c + w_ref[pl.ds(k * D2, half), :]
    y = (jnp.dot(flat_ref[...], W1, preferred_element_type=jnp.float32)
         - jnp.dot(xn_ref[:, :half], w1c, preferred_element_type=jnp.float32)
         + b_ref[...])
    a = y[:, :M]
    g = y[:, M:]
    out_ref[...] = a * (1.0 / (1.0 + jnp.exp(-g)))


def _ff_glu(flat, xn2, W1, b1):
    M_rows, IN = flat.shape
    MB = 512
    OUT = W1.shape[1] // 2
    return pl.pallas_call(
        _ff_glu_body,
        grid=(M_rows // MB,),
        in_specs=[
            pl.BlockSpec((MB, IN), lambda i: (i, 0)),
            pl.BlockSpec((MB, xn2.shape[1]), lambda i: (i, 0)),
            pl.BlockSpec((IN, 2 * OUT), lambda i: (0, 0)),
            pl.BlockSpec((1, 2 * OUT), lambda i: (0, 0)),
        ],
        out_specs=pl.BlockSpec((MB, OUT), lambda i: (i, 0)),
        out_shape=jax.ShapeDtypeStruct((M_rows, OUT), jnp.float32),
    )(flat, xn2, W1, b1.reshape(1, -1))


def kernel(x, features, attn_mask, W1, b1):
    B, N, D = x.shape
    Q = 4
    gk = _make_sc_gather(Q * N, 2 * D, Q * N * K)

    def stage(xq, fq):
        xn, idxg = _knn_topk(xq, fq)
        xn2 = xn.reshape(Q * N, 2 * D)
        idx2d = idxg.reshape(Q * N * K // 128, 128)
        flat = gk(xn2, idx2d).reshape(Q * N, K * 2 * D)
        return _ff_glu(flat, xn2, W1, b1)

    outs = [stage(x[q:q + Q], features[q:q + Q]) for q in range(0, B, Q)]
    return jnp.concatenate(outs, axis=0).reshape(B, N, W1.shape[1] // 2)


# speedup vs baseline: 3.7382x; 3.7382x over previous
import functools

import jax
import jax.numpy as jnp
from jax import lax
from jax.experimental import pallas as pl
from jax.experimental.pallas import tpu as pltpu
from jax.experimental.pallas import tpu_sc as plsc

K = 16
RB = 256
IDX_BITS = 11
IDX_MASK = (1 << IDX_BITS) - 1


def _knn_topk_body(x_ref, f_ref, xn_ref, idx_ref, xcrd_ref, xcrdt_ref):
    b = pl.program_id(0)
    r = pl.program_id(1)
    N = x_ref.shape[1]
    D = x_ref.shape[2]

    @pl.when(r == 0)
    def _prep():
        xb = x_ref[0]
        fmask = f_ref[0] > 0.1
        x_crd = jnp.where(fmask, 0.0, xb)
        x_ftr = jnp.where(fmask, xb, 0.0)
        xc = jnp.concatenate([x_crd, x_ftr], axis=1)
        m = jnp.sum(xc, axis=0, keepdims=True) / N
        var = jnp.sum((xc - m) ** 2, axis=0, keepdims=True) / (N - 1.0)
        s = jnp.sqrt(var)
        xn = jnp.clip((xc - m) / (s + 1e-5), -10.0, 10.0)
        xn_ref[0] = xn
        xcrd_ref[...] = x_crd
        xcrdt_ref[...] = x_crd.T

    base = r * RB
    a = xcrd_ref[pl.ds(base, RB), :]
    xt = xcrdt_ref[...]
    sqa = jnp.sum(a * a, axis=1, keepdims=True)
    sqf = jnp.sum(xt * xt, axis=0, keepdims=True)
    d2 = sqa + sqf - 2.0 * jnp.dot(a, xt, preferred_element_type=jnp.float32)
    key = jnp.maximum(d2, 0.0)

    iota_f = lax.broadcasted_iota(jnp.int32, (RB, N), 1).astype(jnp.float32)
    n_f = jnp.float32(N)
    cols = []
    big = jnp.float32(3e38)
    for _ in range(K):
        mn = jnp.min(key, axis=1, keepdims=True)
        amin = jnp.min(jnp.where(key == mn, iota_f, n_f), axis=1,
                       keepdims=True)
        cols.append(amin)
        key = jnp.where(iota_f == amin, big, key)
    idx_f = jnp.concatenate(cols, axis=1)
    idx_ref[0] = idx_f.astype(jnp.int32) + b * N


def _knn_topk(x, features):
    B, N, D = x.shape
    grid = (B, N // RB)
    xn, idxg = pl.pallas_call(
        _knn_topk_body,
        grid=grid,
        in_specs=[
            pl.BlockSpec((1, N, D), lambda b, r: (b, 0, 0)),
            pl.BlockSpec((1, 1, D), lambda b, r: (b, 0, 0)),
        ],
        out_specs=[
            pl.BlockSpec((1, N, 2 * D), lambda b, r: (b, 0, 0)),
            pl.BlockSpec((1, RB, K), lambda b, r: (b, r, 0)),
        ],
        out_shape=[
            jax.ShapeDtypeStruct((B, N, 2 * D), jnp.float32),
            jax.ShapeDtypeStruct((B, N, K), jnp.int32),
        ],
        scratch_shapes=[
            pltpu.VMEM((N, D), jnp.float32),
            pltpu.VMEM((D, N), jnp.float32),
        ],
    )(x, features.reshape(B, 1, D))
    return xn, idxg


def _make_sc_gather(T, D, Btot):
    info = plsc.get_sparse_core_info()
    NW = info.num_cores * info.num_subcores
    CH = 128
    RPB = 8
    bpw = Btot // NW
    nrounds = bpw // (CH * RPB)
    assert bpw % (CH * RPB) == 0 and nrounds >= 2
    mesh = plsc.VectorSubcoreMesh(core_axis_name="c", subcore_axis_name="s")
    rpr = CH * RPB

    @functools.partial(
        pl.kernel,
        mesh=mesh,
        compiler_params=pltpu.CompilerParams(use_tc_tiling_on_sc=False),
        out_type=jax.ShapeDtypeStruct((Btot, D), jnp.float32),
        scratch_types=[
            pltpu.VMEM((bpw // CH, CH), jnp.int32),
            pltpu.VMEM((2, rpr, D), jnp.float32),
            pltpu.SemaphoreType.DMA,
            pltpu.SemaphoreType.DMA,
        ],
    )
    def gk(table_hbm, idx_hbm, out_hbm, idx_v, rows_v, sem0, sem1):
        wid = lax.axis_index("s") * info.num_cores + lax.axis_index("c")
        row0 = wid * (bpw // CH)
        pltpu.sync_copy(idx_hbm.at[pl.ds(row0, bpw // CH)], idx_v)
        base = wid * bpw
        sems = (sem0, sem1)

        def fire(r, buf):
            sem = sems[buf]
            return [
                pltpu.async_copy(
                    table_hbm.at[idx_v.at[r * RPB + j]],
                    rows_v.at[(buf, pl.ds(j * CH, CH))],
                    sem,
                )
                for j in range(RPB)
            ]

        def drain_and_store(r, buf, copies):
            for c in copies:
                c.wait()
            pltpu.sync_copy(rows_v.at[buf],
                            out_hbm.at[pl.ds(base + r * rpr, rpr)])

        prev = fire(0, 0)
        for r in range(1, nrounds):
            cur = fire(r, r % 2)
            drain_and_store(r - 1, (r - 1) % 2, prev)
            prev = cur
        drain_and_store(nrounds - 1, (nrounds - 1) % 2, prev)

    return gk


def _ff_glu_body(flat_ref, xn_ref, w_ref, b_ref, out_ref):
    W1 = w_ref[...]
    IN = W1.shape[0]
    M = W1.shape[1] // 2
    D2 = xn_ref.shape[1]
    half = D2 // 2
    w1c = w_ref[pl.ds(0, half), :]
    for k in range(1, IN // D2):
        w1c = w1c + w_ref[pl.ds(k * D2, half), :]
    y = (jnp.dot(flat_ref[...], W1, preferred_element_type=jnp.float32)
         - jnp.dot(xn_ref[:, :half], w1c, preferred_element_type=jnp.float32)
         + b_ref[...])
    a = y[:, :M]
    g = y[:, M:]
    out_ref[...] = a * (1.0 / (1.0 + jnp.exp(-g)))


def _ff_glu(flat, xn2, W1, b1):
    M_rows, IN = flat.shape
    MB = 512
    OUT = W1.shape[1] // 2
    return pl.pallas_call(
        _ff_glu_body,
        grid=(M_rows // MB,),
        in_specs=[
            pl.BlockSpec((MB, IN), lambda i: (i, 0)),
            pl.BlockSpec((MB, xn2.shape[1]), lambda i: (i, 0)),
            pl.BlockSpec((IN, 2 * OUT), lambda i: (0, 0)),
            pl.BlockSpec((1, 2 * OUT), lambda i: (0, 0)),
        ],
        out_specs=pl.BlockSpec((MB, OUT), lambda i: (i, 0)),
        out_shape=jax.ShapeDtypeStruct((M_rows, OUT), jnp.float32),
    )(flat, xn2, W1, b1.reshape(1, -1))


def kernel(x, features, attn_mask, W1, b1):
    B, N, D = x.shape
    Q = 4
    gk = _make_sc_gather(Q * N, 2 * D, Q * N * K)

    def stage(xq, fq):
        xn, idxg = _knn_topk(xq, fq)
        xn2 = xn.reshape(Q * N, 2 * D)
        idx2d = idxg.reshape(Q * N * K // 128, 128)
        flat = gk(xn2, idx2d).reshape(Q * N, K * 2 * D)
        return _ff_glu(flat, xn2, W1, b1)

    outs = [stage(x[q:q + Q], features[q:q + Q]) for q in range(0, B, Q)]
    return jnp.concatenate(outs, axis=0).reshape(B, N, W1.shape[1] // 2)
